# Initial kernel scaffold; baseline (speedup 1.0000x reference)
import jax
import jax.numpy as jnp
from jax.experimental import pallas as pl


def kernel(adj, x, w1, b1, w2, b2):
    raise NotImplementedError("write your pallas kernel here")


# speedup vs baseline: 1.3664x; 1.3664x over previous
import functools

import jax
import jax.numpy as jnp
from jax.experimental import pallas as pl
from jax.experimental.pallas import tpu as pltpu

_TM = 512


def _vmem_limit():
    return 100 << 20


def _xw_kernel(x_ref, w_ref, out_ref):
    x = x_ref[...].astype(jnp.bfloat16)
    w = w_ref[...].astype(jnp.bfloat16)
    out_ref[...] = jnp.dot(x, w, preferred_element_type=jnp.float32
                           ).astype(out_ref.dtype)


def _feature_transform(x, w, *, tm):
    n, c_in = x.shape
    c_out = w.shape[1]
    return pl.pallas_call(
        _xw_kernel,
        out_shape=jax.ShapeDtypeStruct((n, c_out), jnp.bfloat16),
        grid=(n // tm,),
        in_specs=[pl.BlockSpec((tm, c_in), lambda i: (i, 0)),
                  pl.BlockSpec((c_in, c_out), lambda i: (0, 0))],
        out_specs=pl.BlockSpec((tm, c_out), lambda i: (i, 0)),
        compiler_params=pltpu.CompilerParams(
            dimension_semantics=("parallel",),
            vmem_limit_bytes=_vmem_limit()),
        cost_estimate=pl.CostEstimate(
            flops=int(2 * n * c_in * c_out), transcendentals=0,
            bytes_accessed=int(x.size * 4 + w.size * 4 + n * c_out * 2)),
    )(x, w)


def _agg1_kernel(adj_ref, u_ref, b1_ref, w2_ref, out_ref):
    a = adj_ref[...].astype(jnp.bfloat16)
    h = jnp.dot(a, u_ref[...], preferred_element_type=jnp.float32)
    h = jnp.maximum(h + b1_ref[...], 0.0).astype(jnp.bfloat16)
    v = jnp.dot(h, w2_ref[...].astype(jnp.bfloat16),
                preferred_element_type=jnp.float32)
    out_ref[...] = v.astype(out_ref.dtype)


def _layer1(adj, u, b1, w2, *, tm):
    n = adj.shape[0]
    c_hid = u.shape[1]
    c_out = w2.shape[1]
    flops = 2 * n * n * c_hid + 2 * n * c_hid * c_out
    bytes_accessed = int(adj.size * 4 + u.size * 2 + n * c_out * 2)
    return pl.pallas_call(
        _agg1_kernel,
        out_shape=jax.ShapeDtypeStruct((n, c_out), jnp.bfloat16),
        grid=(n // tm,),
        in_specs=[pl.BlockSpec((tm, n), lambda i: (i, 0)),
                  pl.BlockSpec((n, c_hid), lambda i: (0, 0)),
                  pl.BlockSpec((1, c_hid), lambda i: (0, 0)),
                  pl.BlockSpec((c_hid, c_out), lambda i: (0, 0))],
        out_specs=pl.BlockSpec((tm, c_out), lambda i: (i, 0)),
        compiler_params=pltpu.CompilerParams(
            dimension_semantics=("parallel",),
            vmem_limit_bytes=_vmem_limit()),
        cost_estimate=pl.CostEstimate(flops=int(flops), transcendentals=0,
                                      bytes_accessed=bytes_accessed),
    )(adj, u, b1, w2)


def _agg2_kernel(adj_ref, v_ref, b2_ref, out_ref):
    a = adj_ref[...].astype(jnp.bfloat16)
    z = jnp.dot(a, v_ref[...], preferred_element_type=jnp.float32)
    out_ref[...] = (z + b2_ref[...]).astype(out_ref.dtype)


def _layer2(adj, v, b2, *, tm):
    n = adj.shape[0]
    c_out = v.shape[1]
    flops = 2 * n * n * c_out
    bytes_accessed = int(adj.size * 4 + v.size * 2 + n * c_out * 4)
    return pl.pallas_call(
        _agg2_kernel,
        out_shape=jax.ShapeDtypeStruct((n, c_out), jnp.float32),
        grid=(n // tm,),
        in_specs=[pl.BlockSpec((tm, n), lambda i: (i, 0)),
                  pl.BlockSpec((n, c_out), lambda i: (0, 0)),
                  pl.BlockSpec((1, c_out), lambda i: (0, 0))],
        out_specs=pl.BlockSpec((tm, c_out), lambda i: (i, 0)),
        compiler_params=pltpu.CompilerParams(
            dimension_semantics=("parallel",),
            vmem_limit_bytes=_vmem_limit()),
        cost_estimate=pl.CostEstimate(flops=int(flops), transcendentals=0,
                                      bytes_accessed=bytes_accessed),
    )(adj, v, b2)


def kernel(adj, x, w1, b1, w2, b2):
    n = x.shape[0]
    u = _feature_transform(x, w1, tm=_TM)
    v = _layer1(adj, u, b1, w2, tm=_TM)
    z = _layer2(adj, v, b2, tm=_TM)
    return z
